# baseline (speedup 1.0000x reference)
import functools

import jax
import jax.numpy as jnp
from jax import lax
from jax.experimental import pallas as pl
from jax.experimental.pallas import tpu as pltpu
from jax.experimental.pallas import tpu_sc as plsc

N = 10000
E = 320000
D = 128
NC = 2
NS = 16
NW = NC * NS
EPT = E // NW
CH = 80
NCHUNK = EPT // CH
CTILES = 10
RPT = N // CTILES
ZROWS = 200
LANES = 16


DEG_LAG = 4


def _sc_agg_body(src_hbm, dst_hbm, feat_hbm, psum_hbm, degp_hbm,
                 sidx_f, didx_f, idx_a, idx_b, idd_v, idd_w, rows_a, rows_b,
                 dzbuf, ones_v, shared_sum, shared_deg, sem_a, sem_b, dsem):
    c = lax.axis_index("c")
    s = lax.axis_index("s")
    wid = c * NS + s
    zero16 = jnp.zeros((LANES,), jnp.float32)
    ones16 = jnp.ones((LANES,), jnp.float32)

    pltpu.sync_copy(src_hbm.at[pl.ds(wid * EPT, EPT)], sidx_f)
    pltpu.sync_copy(dst_hbm.at[pl.ds(wid * EPT, EPT)], didx_f)

    def zrow(r, carry):
        for j in range(D // LANES):
            rows_a[r, pl.ds(j * LANES, LANES)] = zero16
        return carry
    lax.fori_loop(0, CH, zrow, 0)

    def zdz(i, carry):
        dzbuf[pl.ds(i * LANES, LANES)] = zero16
        return carry
    lax.fori_loop(0, RPT // LANES, zdz, 0)

    for q in range(CH // LANES):
        ones_v[pl.ds(q * LANES, LANES)] = ones16

    row0 = s * RPT

    @pl.when(s < CTILES)
    def _():
        def zshared(k, carry):
            pltpu.sync_copy(rows_a, shared_sum.at[pl.ds(row0 + k * CH, CH)])
            return carry
        lax.fori_loop(0, RPT // CH, zshared, 0)
        rem = RPT - (RPT // CH) * CH
        pltpu.sync_copy(
            rows_a.at[pl.ds(0, rem)],
            shared_sum.at[pl.ds(row0 + (RPT // CH) * CH, rem)])
        pltpu.sync_copy(dzbuf, shared_deg.at[pl.ds(row0, RPT)])

    def cpidx(src_ref, t, ibuf):
        for q in range(CH // LANES):
            ibuf[pl.ds(q * LANES, LANES)] = (
                src_ref[pl.ds(t * CH + q * LANES, LANES)])

    def gstart(t, buf, sem, ibuf):
        cpidx(sidx_f, t, ibuf)
        pltpu.async_copy(feat_hbm.at[ibuf], buf, sem)

    def gwait(t, buf, sem, ibuf):
        pltpu.make_async_copy(feat_hbm.at[ibuf], buf, sem).wait()

    def scat(t, buf, idd):
        cpidx(didx_f, t, idd)
        pltpu.sync_copy(buf, shared_sum.at[idd], add=True)
        pltpu.sync_copy(ones_v, shared_deg.at[idd], add=True)

    gstart(0, rows_a, sem_a, idx_a)
    plsc.subcore_barrier()

    def pair(i, carry):
        t0 = 2 * i
        t1 = t0 + 1
        gwait(t0, rows_a, sem_a, idx_a)
        gstart(t1, rows_b, sem_b, idx_b)
        scat(t0, rows_a, idd_v)
        gwait(t1, rows_b, sem_b, idx_b)
        gstart(t0 + 2, rows_a, sem_a, idx_a)
        scat(t1, rows_b, idd_w)
        return carry
    lax.fori_loop(0, (NCHUNK - 1) // 2, pair, 0)

    t_last = NCHUNK - 1
    gwait(t_last, rows_a, sem_a, idx_a)
    scat(t_last, rows_a, idd_v)

    plsc.subcore_barrier()

    @pl.when(s < CTILES)
    def _():
        def cout(k, carry):
            r = row0 + k * CH
            pltpu.sync_copy(shared_sum.at[pl.ds(r, CH)], rows_a)
            pltpu.sync_copy(rows_a, psum_hbm.at[c].at[pl.ds(r, CH)])
            return carry
        lax.fori_loop(0, RPT // CH, cout, 0)
        rem = RPT - (RPT // CH) * CH
        r_rem = row0 + (RPT // CH) * CH
        pltpu.sync_copy(shared_sum.at[pl.ds(r_rem, rem)],
                        rows_a.at[pl.ds(0, rem)])
        pltpu.sync_copy(rows_a.at[pl.ds(0, rem)],
                        psum_hbm.at[c].at[pl.ds(r_rem, rem)])
        pltpu.sync_copy(shared_deg.at[pl.ds(row0, RPT)], dzbuf)
        pltpu.sync_copy(dzbuf, degp_hbm.at[pl.ds(c * N + row0, RPT)])


@jax.jit
def _sc_agg(src, dst, features):
    mesh = plsc.VectorSubcoreMesh(core_axis_name="c", subcore_axis_name="s")
    f = pl.kernel(
        _sc_agg_body,
        mesh=mesh,
        out_type=[
            jax.ShapeDtypeStruct((NC, N, D), jnp.float32),
            jax.ShapeDtypeStruct((NC * N,), jnp.float32),
        ],
        scratch_types=[
            pltpu.VMEM((EPT,), jnp.int32),
            pltpu.VMEM((EPT,), jnp.int32),
            pltpu.VMEM((CH,), jnp.int32),
            pltpu.VMEM((CH,), jnp.int32),
            pltpu.VMEM((CH,), jnp.int32),
            pltpu.VMEM((CH,), jnp.int32),
            pltpu.VMEM((CH, D), jnp.float32),
            pltpu.VMEM((CH, D), jnp.float32),
            pltpu.VMEM((RPT,), jnp.float32),
            pltpu.VMEM((CH,), jnp.float32),
            pltpu.VMEM_SHARED((N, D), jnp.float32),
            pltpu.VMEM_SHARED((N,), jnp.float32),
            pltpu.SemaphoreType.DMA,
            pltpu.SemaphoreType.DMA,
            pltpu.SemaphoreType.DMA,
        ],
    )
    return f(src, dst, features)


ROWS_BLK = 1000


def _tc_self_body(f_ref, w_ref, o_ref):
    o_ref[...] = jnp.dot(f_ref[...], w_ref[...],
                         preferred_element_type=jnp.float32)


def _tc_self(features, weight):
    return pl.pallas_call(
        _tc_self_body,
        grid=(N // ROWS_BLK,),
        in_specs=[
            pl.BlockSpec((ROWS_BLK, D), lambda i: (i, 0)),
            pl.BlockSpec((D, D), lambda i: (0, 0)),
        ],
        out_specs=pl.BlockSpec((ROWS_BLK, D), lambda i: (i, 0)),
        out_shape=jax.ShapeDtypeStruct((N, D), jnp.float32),
    )(features, weight)


def _tc_final_body(s_ref, p_ref, dp_ref, w_ref, o_ref):
    ssum = p_ref[0] + p_ref[1]
    deg = jnp.sum(dp_ref[...], axis=1)
    inv = 1.0 / jnp.maximum(deg, 1.0)
    neigh = ssum * inv[:, None]
    acc = s_ref[...] + jnp.dot(neigh, w_ref[...],
                               preferred_element_type=jnp.float32)
    o_ref[...] = jnp.maximum(acc, 0.0)


def _tc_final(selfmm, psum, degp, weight):
    return pl.pallas_call(
        _tc_final_body,
        grid=(N // ROWS_BLK,),
        in_specs=[
            pl.BlockSpec((ROWS_BLK, D), lambda i: (i, 0)),
            pl.BlockSpec((NC, ROWS_BLK, D), lambda i: (0, i, 0)),
            pl.BlockSpec((ROWS_BLK, NC), lambda i: (i, 0)),
            pl.BlockSpec((D, D), lambda i: (0, 0)),
        ],
        out_specs=pl.BlockSpec((ROWS_BLK, D), lambda i: (i, 0)),
        out_shape=jax.ShapeDtypeStruct((N, D), jnp.float32),
    )(selfmm, psum, degp, weight)


@jax.jit
def kernel(features, adj, weight):
    adj32 = adj.astype(jnp.int32)
    src = adj32[0]
    dst = adj32[1]
    selfmm = _tc_self(features, weight[:D])
    psum, degp = _sc_agg(src, dst, features)
    degp_t = degp.reshape(NC, N).T
    return _tc_final(selfmm, psum, degp_t, weight[D:])


# speedup vs baseline: 1.4060x; 1.4060x over previous
import functools

import jax
import jax.numpy as jnp
from jax import lax
from jax.experimental import pallas as pl
from jax.experimental.pallas import tpu as pltpu
from jax.experimental.pallas import tpu_sc as plsc

N = 10000
E = 320000
D = 128
NC = 2
NS = 16
NW = NC * NS
EPT = E // NW
CH = 80
NCHUNK = EPT // CH
CTILES = 10
RPT = N // CTILES
ZROWS = 200
LANES = 16


NBUF = 3
NTRIPLE = 40


def _sc_agg_body(combo_hbm, feat_hbm, psum_hbm, degp_hbm,
                 combo_v, idx_0, idx_1, idx_2, idd_0, idd_1, idd_2,
                 rows_0, rows_1, rows_2, dzbuf, ones_v,
                 shared_sum, shared_deg, sem_0, sem_1, sem_2):
    c = lax.axis_index("c")
    s = lax.axis_index("s")
    wid = c * NS + s
    zero16 = jnp.zeros((LANES,), jnp.float32)
    ones16 = jnp.ones((LANES,), jnp.float32)
    idx_b = [idx_0, idx_1, idx_2]
    idd_b = [idd_0, idd_1, idd_2]
    rows_b = [rows_0, rows_1, rows_2]
    sem_b = [sem_0, sem_1, sem_2]

    pltpu.sync_copy(combo_hbm.at[pl.ds(wid * EPT, EPT)], combo_v)

    def zrow(r, carry):
        for j in range(D // LANES):
            rows_0[r, pl.ds(j * LANES, LANES)] = zero16
        return carry
    lax.fori_loop(0, CH, zrow, 0)

    def zdz(i, carry):
        dzbuf[pl.ds(i * LANES, LANES)] = zero16
        return carry
    lax.fori_loop(0, RPT // LANES, zdz, 0)

    for q in range(CH // LANES):
        ones_v[pl.ds(q * LANES, LANES)] = ones16

    row0 = s * RPT

    @pl.when(s < CTILES)
    def _():
        def zshared(k, carry):
            pltpu.sync_copy(rows_0, shared_sum.at[pl.ds(row0 + k * CH, CH)])
            return carry
        lax.fori_loop(0, RPT // CH, zshared, 0)
        rem = RPT - (RPT // CH) * CH
        pltpu.sync_copy(
            rows_0.at[pl.ds(0, rem)],
            shared_sum.at[pl.ds(row0 + (RPT // CH) * CH, rem)])
        pltpu.sync_copy(dzbuf, shared_deg.at[pl.ds(row0, RPT)])

    def gfire(t, b):
        for q in range(CH // LANES):
            v = combo_v[pl.ds(t * CH + q * LANES, LANES)]
            idx_b[b][pl.ds(q * LANES, LANES)] = v & 0xFFFF
            idd_b[b][pl.ds(q * LANES, LANES)] = lax.shift_right_logical(v, 16)
        pltpu.async_copy(feat_hbm.at[idx_b[b]], rows_b[b], sem_b[b])

    def gwait(b):
        pltpu.make_async_copy(feat_hbm.at[idx_b[b]], rows_b[b],
                              sem_b[b]).wait()

    def scat(b):
        pltpu.sync_copy(rows_b[b], shared_sum.at[idd_b[b]], add=True)
        pltpu.sync_copy(ones_v, shared_deg.at[idd_b[b]], add=True)

    for b in range(NBUF):
        gfire(b, b)
    plsc.subcore_barrier()

    def triple(i, carry):
        t0 = NBUF * i
        for b in range(NBUF):
            gwait(b)
            scat(b)
            gfire(t0 + b + NBUF, b)
        return carry
    lax.fori_loop(0, NTRIPLE, triple, 0)

    gwait(0)
    scat(0)
    gfire(NBUF * NTRIPLE + NBUF, 0)
    gwait(1)
    scat(1)
    gfire(NBUF * NTRIPLE + NBUF + 1, 1)
    gwait(2)
    scat(2)
    gwait(0)
    scat(0)
    gwait(1)
    scat(1)

    plsc.subcore_barrier()

    @pl.when(s < CTILES)
    def _():
        def cout(k, carry):
            r = row0 + k * CH
            pltpu.sync_copy(shared_sum.at[pl.ds(r, CH)], rows_0)
            pltpu.sync_copy(rows_0, psum_hbm.at[c].at[pl.ds(r, CH)])
            return carry
        lax.fori_loop(0, RPT // CH, cout, 0)
        rem = RPT - (RPT // CH) * CH
        r_rem = row0 + (RPT // CH) * CH
        pltpu.sync_copy(shared_sum.at[pl.ds(r_rem, rem)],
                        rows_0.at[pl.ds(0, rem)])
        pltpu.sync_copy(rows_0.at[pl.ds(0, rem)],
                        psum_hbm.at[c].at[pl.ds(r_rem, rem)])
        pltpu.sync_copy(shared_deg.at[pl.ds(row0, RPT)], dzbuf)
        pltpu.sync_copy(dzbuf, degp_hbm.at[pl.ds(c * N + row0, RPT)])


@jax.jit
def _sc_agg(combo, features):
    mesh = plsc.VectorSubcoreMesh(core_axis_name="c", subcore_axis_name="s")
    f = pl.kernel(
        _sc_agg_body,
        mesh=mesh,
        out_type=[
            jax.ShapeDtypeStruct((NC, N, D), jnp.float32),
            jax.ShapeDtypeStruct((NC * N,), jnp.float32),
        ],
        scratch_types=[
            pltpu.VMEM((EPT,), jnp.int32),
            pltpu.VMEM((CH,), jnp.int32),
            pltpu.VMEM((CH,), jnp.int32),
            pltpu.VMEM((CH,), jnp.int32),
            pltpu.VMEM((CH,), jnp.int32),
            pltpu.VMEM((CH,), jnp.int32),
            pltpu.VMEM((CH,), jnp.int32),
            pltpu.VMEM((CH, D), jnp.float32),
            pltpu.VMEM((CH, D), jnp.float32),
            pltpu.VMEM((CH, D), jnp.float32),
            pltpu.VMEM((RPT,), jnp.float32),
            pltpu.VMEM((CH,), jnp.float32),
            pltpu.VMEM_SHARED((N, D), jnp.float32),
            pltpu.VMEM_SHARED((N,), jnp.float32),
            pltpu.SemaphoreType.DMA,
            pltpu.SemaphoreType.DMA,
            pltpu.SemaphoreType.DMA,
        ],
    )
    return f(combo, features)


ROWS_BLK = 1000


def _tc_self_body(f_ref, w_ref, o_ref):
    o_ref[...] = jnp.dot(f_ref[...], w_ref[...],
                         preferred_element_type=jnp.float32)


def _tc_self(features, weight):
    return pl.pallas_call(
        _tc_self_body,
        grid=(N // ROWS_BLK,),
        in_specs=[
            pl.BlockSpec((ROWS_BLK, D), lambda i: (i, 0)),
            pl.BlockSpec((D, D), lambda i: (0, 0)),
        ],
        out_specs=pl.BlockSpec((ROWS_BLK, D), lambda i: (i, 0)),
        out_shape=jax.ShapeDtypeStruct((N, D), jnp.float32),
    )(features, weight)


def _tc_final_body(s_ref, p_ref, dp_ref, w_ref, o_ref):
    ssum = p_ref[0] + p_ref[1]
    deg = jnp.sum(dp_ref[...], axis=1)
    inv = 1.0 / jnp.maximum(deg, 1.0)
    neigh = ssum * inv[:, None]
    acc = s_ref[...] + jnp.dot(neigh, w_ref[...],
                               preferred_element_type=jnp.float32)
    o_ref[...] = jnp.maximum(acc, 0.0)


def _tc_final(selfmm, psum, degp, weight):
    return pl.pallas_call(
        _tc_final_body,
        grid=(N // ROWS_BLK,),
        in_specs=[
            pl.BlockSpec((ROWS_BLK, D), lambda i: (i, 0)),
            pl.BlockSpec((NC, ROWS_BLK, D), lambda i: (0, i, 0)),
            pl.BlockSpec((ROWS_BLK, NC), lambda i: (i, 0)),
            pl.BlockSpec((D, D), lambda i: (0, 0)),
        ],
        out_specs=pl.BlockSpec((ROWS_BLK, D), lambda i: (i, 0)),
        out_shape=jax.ShapeDtypeStruct((N, D), jnp.float32),
    )(selfmm, psum, degp, weight)


@jax.jit
def kernel(features, adj, weight):
    adj32 = adj.astype(jnp.int32)
    combo = adj32[0] | (adj32[1] << 16)
    selfmm = _tc_self(features, weight[:D])
    psum, degp = _sc_agg(combo, features)
    degp_t = degp.reshape(NC, N).T
    return _tc_final(selfmm, psum, degp_t, weight[D:])


# speedup vs baseline: 1.4263x; 1.0144x over previous
import jax
import jax.numpy as jnp
from jax import lax
from jax.experimental import pallas as pl
from jax.experimental.pallas import tpu as pltpu
from jax.experimental.pallas import tpu_sc as plsc

N = 10000
E = 320000
D = 128
NC = 2
NS = 16
NW = NC * NS
EPT = E // NW
CH = 80
NCHUNK = EPT // CH
CTILES = 10
RPT = N // CTILES
LANES = 16
NBUF = 4
NLOOP = NCHUNK // NBUF


def _sc_agg_body(combo_hbm, feat_hbm, psum_hbm, degp_hbm,
                 cb_0, cb_1, cb_2, cb_3, idx_0, idx_1, idx_2, idx_3,
                 idd_0, idd_1, idd_2, idd_3,
                 rows_0, rows_1, rows_2, rows_3, dzbuf, ones_v,
                 shared_sum, shared_deg,
                 gsem_0, gsem_1, gsem_2, gsem_3,
                 csem_0, csem_1, csem_2, csem_3):
    c = lax.axis_index("c")
    s = lax.axis_index("s")
    wid = c * NS + s
    zero16 = jnp.zeros((LANES,), jnp.float32)
    ones16 = jnp.ones((LANES,), jnp.float32)
    cb_b = [cb_0, cb_1, cb_2, cb_3]
    idx_b = [idx_0, idx_1, idx_2, idx_3]
    idd_b = [idd_0, idd_1, idd_2, idd_3]
    rows_b = [rows_0, rows_1, rows_2, rows_3]
    gsem_b = [gsem_0, gsem_1, gsem_2, gsem_3]
    csem_b = [csem_0, csem_1, csem_2, csem_3]
    ebase = wid * EPT

    def cfire(t, b):
        pltpu.async_copy(combo_hbm.at[pl.ds(ebase + t * CH, CH)], cb_b[b],
                         csem_b[b])

    def cwait(t, b):
        pltpu.make_async_copy(combo_hbm.at[pl.ds(ebase + t * CH, CH)],
                              cb_b[b], csem_b[b]).wait()

    def unpack(b):
        for q in range(CH // LANES):
            v = cb_b[b][pl.ds(q * LANES, LANES)]
            idx_b[b][pl.ds(q * LANES, LANES)] = v & 0xFFFF
            idd_b[b][pl.ds(q * LANES, LANES)] = lax.shift_right_logical(v, 16)

    def gfire(b):
        pltpu.async_copy(feat_hbm.at[idx_b[b]], rows_b[b], gsem_b[b])

    def gwait(b):
        pltpu.make_async_copy(feat_hbm.at[idx_b[b]], rows_b[b],
                              gsem_b[b]).wait()

    def scat(b):
        pltpu.sync_copy(rows_b[b], shared_sum.at[idd_b[b]], add=True)
        pltpu.sync_copy(ones_v, shared_deg.at[idd_b[b]], add=True)

    for b in range(NBUF):
        cfire(b, b)

    def zrow(r, carry):
        for j in range(D // LANES):
            rows_0[r, pl.ds(j * LANES, LANES)] = zero16
        return carry
    lax.fori_loop(0, CH, zrow, 0)

    def zdz(i, carry):
        dzbuf[pl.ds(i * LANES, LANES)] = zero16
        return carry
    lax.fori_loop(0, RPT // LANES, zdz, 0)

    for q in range(CH // LANES):
        ones_v[pl.ds(q * LANES, LANES)] = ones16

    row0 = s * RPT

    @pl.when(s < CTILES)
    def _():
        def zshared(k, carry):
            pltpu.sync_copy(rows_0, shared_sum.at[pl.ds(row0 + k * CH, CH)])
            return carry
        lax.fori_loop(0, RPT // CH, zshared, 0)
        rem = RPT - (RPT // CH) * CH
        pltpu.sync_copy(
            rows_0.at[pl.ds(0, rem)],
            shared_sum.at[pl.ds(row0 + (RPT // CH) * CH, rem)])
        pltpu.sync_copy(dzbuf, shared_deg.at[pl.ds(row0, RPT)])

    for b in range(NBUF):
        cwait(b, b)
        unpack(b)
        gfire(b)
        cfire(b + NBUF, b)

    plsc.subcore_barrier()

    def step(i, carry):
        t0 = NBUF * i
        for b in range(NBUF):
            t = t0 + b
            gwait(b)
            scat(b)
            tn = t + NBUF

            @pl.when(tn < NCHUNK)
            def _():
                cwait(tn, b)
                unpack(b)
                gfire(b)
                tp = t + 2 * NBUF

                @pl.when(tp < NCHUNK)
                def _():
                    cfire(tp, b)
        return carry
    lax.fori_loop(0, NLOOP, step, 0)

    gwait(0)
    scat(0)

    plsc.subcore_barrier()

    @pl.when(s < CTILES)
    def _():
        def cout(k, carry):
            r = row0 + k * CH
            pltpu.sync_copy(shared_sum.at[pl.ds(r, CH)], rows_0)
            pltpu.sync_copy(rows_0, psum_hbm.at[c].at[pl.ds(r, CH)])
            return carry
        lax.fori_loop(0, RPT // CH, cout, 0)
        rem = RPT - (RPT // CH) * CH
        r_rem = row0 + (RPT // CH) * CH
        pltpu.sync_copy(shared_sum.at[pl.ds(r_rem, rem)],
                        rows_0.at[pl.ds(0, rem)])
        pltpu.sync_copy(rows_0.at[pl.ds(0, rem)],
                        psum_hbm.at[c].at[pl.ds(r_rem, rem)])
        pltpu.sync_copy(shared_deg.at[pl.ds(row0, RPT)], dzbuf)
        pltpu.sync_copy(dzbuf, degp_hbm.at[pl.ds(c * N + row0, RPT)])


@jax.jit
def _sc_agg(combo, features):
    mesh = plsc.VectorSubcoreMesh(core_axis_name="c", subcore_axis_name="s")
    f = pl.kernel(
        _sc_agg_body,
        mesh=mesh,
        out_type=[
            jax.ShapeDtypeStruct((NC, N, D), jnp.float32),
            jax.ShapeDtypeStruct((NC * N,), jnp.float32),
        ],
        scratch_types=(
            [pltpu.VMEM((CH,), jnp.int32) for _ in range(3 * NBUF)]
            + [pltpu.VMEM((CH, D), jnp.float32) for _ in range(NBUF)]
            + [
                pltpu.VMEM((RPT,), jnp.float32),
                pltpu.VMEM((CH,), jnp.float32),
                pltpu.VMEM_SHARED((N, D), jnp.float32),
                pltpu.VMEM_SHARED((N,), jnp.float32),
            ]
            + [pltpu.SemaphoreType.DMA for _ in range(2 * NBUF)]
        ),
    )
    return f(combo, features)


ROWS_BLK = 1000


def _tc_self_body(f_ref, w_ref, o_ref):
    o_ref[...] = jnp.dot(f_ref[...], w_ref[...],
                         preferred_element_type=jnp.float32)


def _tc_self(features, weight):
    return pl.pallas_call(
        _tc_self_body,
        grid=(N // ROWS_BLK,),
        in_specs=[
            pl.BlockSpec((ROWS_BLK, D), lambda i: (i, 0)),
            pl.BlockSpec((D, D), lambda i: (0, 0)),
        ],
        out_specs=pl.BlockSpec((ROWS_BLK, D), lambda i: (i, 0)),
        out_shape=jax.ShapeDtypeStruct((N, D), jnp.float32),
    )(features, weight)


def _tc_final_body(s_ref, p_ref, dp_ref, w_ref, o_ref):
    ssum = p_ref[0] + p_ref[1]
    deg = jnp.sum(dp_ref[...], axis=1)
    inv = 1.0 / jnp.maximum(deg, 1.0)
    neigh = ssum * inv[:, None]
    acc = s_ref[...] + jnp.dot(neigh, w_ref[...],
                               preferred_element_type=jnp.float32)
    o_ref[...] = jnp.maximum(acc, 0.0)


def _tc_final(selfmm, psum, degp, weight):
    return pl.pallas_call(
        _tc_final_body,
        grid=(N // ROWS_BLK,),
        in_specs=[
            pl.BlockSpec((ROWS_BLK, D), lambda i: (i, 0)),
            pl.BlockSpec((NC, ROWS_BLK, D), lambda i: (0, i, 0)),
            pl.BlockSpec((ROWS_BLK, NC), lambda i: (i, 0)),
            pl.BlockSpec((D, D), lambda i: (0, 0)),
        ],
        out_specs=pl.BlockSpec((ROWS_BLK, D), lambda i: (i, 0)),
        out_shape=jax.ShapeDtypeStruct((N, D), jnp.float32),
    )(selfmm, psum, degp, weight)


@jax.jit
def kernel(features, adj, weight):
    adj32 = adj.astype(jnp.int32)
    combo = adj32[0] | (adj32[1] << 16)
    selfmm = _tc_self(features, weight[:D])
    psum, degp = _sc_agg(combo, features)
    degp_t = degp.reshape(NC, N).T
    return _tc_final(selfmm, psum, degp_t, weight[D:])
